# baseline (speedup 1.0000x reference)
import functools

import jax
import jax.numpy as jnp
from jax import lax
from jax.experimental import pallas as pl
from jax.experimental.pallas import tpu as pltpu
from jax.experimental.pallas import tpu_sc as plsc

_NT = 12048
_D = 256
_DH = 128
_NC = 2
_NS = 16
_L = 16
_G = 128
_GPC = 2
_CH = _G * _GPC
_RGRP = 8
_RPT = 752
_RB = 2008


def _codemap_body(f2_ref, wct_ref, bc_ref, o_ref):
    o_ref[...] = (
        jnp.dot(f2_ref[...], wct_ref[...], preferred_element_type=jnp.float32)
        + bc_ref[...]
    )


def _mlp_body(x_ref, w1t_ref, b1_ref, w2t_ref, b2_ref, o_ref):
    h = jnp.dot(x_ref[...], w1t_ref[...], preferred_element_type=jnp.float32)
    h = h + b1_ref[...]
    h = jnp.where(h > 0, h, 0.1 * h)
    o = jnp.dot(h, w2t_ref[...], preferred_element_type=jnp.float32)
    o = o + b2_ref[...]
    o_ref[0] = o[:, :_DH]
    o_ref[1] = o[:, _DH:]


def _sc_body(nb, h_hbm, rowi_hbm, coli_hbm, vali_hbm, out_hbm, y_hbm,
             rowv, colv, valv, rows_v, acc, isem, gsem, ssem):
    c = lax.axis_index("c")
    s = lax.axis_index("s")
    cbias = c * _NT
    gbase = s * nb
    zerov = jnp.zeros((_L,), jnp.float32)

    def _zero_acc():
        zbuf = rows_v.at[0]

        def zbody(r, carry):
            for fb in range(_DH // _L):
                zbuf[r, pl.ds(fb * _L, _L)] = zerov
            return carry
        lax.fori_loop(0, _G, zbody, 0)
        for off in range(0, _RPT, _G):
            n = min(_G, _RPT - off)
            pltpu.sync_copy(zbuf.at[pl.ds(0, n)],
                            acc.at[pl.ds(s * _RPT + off, n)])

        @pl.when(s == _NS - 1)
        def _():
            pltpu.sync_copy(zbuf.at[pl.ds(0, _NT - _NS * _RPT)],
                            acc.at[pl.ds(_NS * _RPT, _NT - _NS * _RPT)])

    def _pass(table_hbm, dst_hbm):
        def idx_descs(g, slot):
            e0 = (gbase + g) * _G
            return (
                pltpu.make_async_copy(coli_hbm.at[pl.ds(e0, _G)],
                                      colv.at[slot], isem),
                pltpu.make_async_copy(vali_hbm.at[pl.ds(e0, _G)],
                                      valv.at[slot], isem),
                pltpu.make_async_copy(rowi_hbm.at[pl.ds(e0, _G)],
                                      rowv.at[slot], isem),
            )

        def gat_desc(slot, b):
            return pltpu.make_async_copy(table_hbm.at[colv.at[slot]],
                                         rows_v.at[b], gsem)

        for d in idx_descs(0, 0):
            d.start()

        def group(g, carry):
            gi = lax.rem(g, 3)
            gp1 = lax.rem(g + 1, 3)
            gm1 = lax.rem(g + 2, 3)
            b = lax.rem(g, 2)
            bm1 = lax.rem(g + 1, 2)

            for d in idx_descs(g, gi):
                d.wait()

            def bias(t, cc):
                colv[gi, pl.ds(t * _L, _L)] = (
                    colv[gi, pl.ds(t * _L, _L)] + cbias)
                return cc
            lax.fori_loop(0, _G // _L, bias, 0)

            @pl.when(g >= 2)
            def _():
                pltpu.make_async_copy(
                    rows_v.at[b], acc.at[rowv.at[gp1]], ssem).wait()

            gat_desc(gi, b).start()

            @pl.when(g >= 1)
            def _():
                gat_desc(gm1, bm1).wait()

            @pl.when(g < nb - 1)
            def _():
                for d in idx_descs(g + 1, gp1):
                    d.start()

            @pl.when(g >= 1)
            def _():
                buf = rows_v.at[bm1]

                def scale(t, cc):
                    val16 = valv[gm1, pl.ds(t * _L, _L)]
                    for u in range(_L):
                        r = t * _L + u
                        v = val16[u]
                        for fb in range(_DH // _L):
                            buf[r, pl.ds(fb * _L, _L)] = (
                                buf[r, pl.ds(fb * _L, _L)] * v)
                    return cc
                lax.fori_loop(0, _G // _L, scale, 0)
                pltpu.async_copy(rows_v.at[bm1], acc.at[rowv.at[gm1]], ssem,
                                 add=True)
            return carry
        lax.fori_loop(0, nb, group, 0)

        gl = lax.rem(nb - 1, 3)
        bl = lax.rem(nb - 1, 2)
        gat_desc(gl, bl).wait()
        bufl = rows_v.at[bl]

        def scale_l(t, cc):
            val16 = valv[gl, pl.ds(t * _L, _L)]
            for u in range(_L):
                r = t * _L + u
                v = val16[u]
                for fb in range(_DH // _L):
                    bufl[r, pl.ds(fb * _L, _L)] = (
                        bufl[r, pl.ds(fb * _L, _L)] * v)
            return cc
        lax.fori_loop(0, _G // _L, scale_l, 0)
        pltpu.async_copy(rows_v.at[bl], acc.at[rowv.at[gl]], ssem, add=True)
        pltpu.make_async_copy(
            rows_v.at[lax.rem(nb, 2)],
            acc.at[rowv.at[lax.rem(nb + 1, 3)]], ssem).wait()
        pltpu.make_async_copy(rows_v.at[bl], acc.at[rowv.at[gl]], ssem).wait()

        plsc.subcore_barrier()
        pltpu.sync_copy(acc.at[pl.ds(s * _RPT, _RPT)],
                        dst_hbm.at[pl.ds(cbias + s * _RPT, _RPT)])

        @pl.when(s == _NS - 1)
        def _():
            rem = _NT - _NS * _RPT
            pltpu.sync_copy(acc.at[pl.ds(_NS * _RPT, rem)],
                            dst_hbm.at[pl.ds(cbias + _NS * _RPT, rem)])

    _zero_acc()
    plsc.subcore_barrier()
    _pass(h_hbm, y_hbm)
    _zero_acc()
    plsc.subcore_barrier()
    _pass(y_hbm, out_hbm)


def kernel(seq_a, adj_indices, adj_values, node_emb, init_fea2, Wc, bc,
           W1, b1, W2, b2):
    del seq_a

    cm = pl.pallas_call(
        _codemap_body,
        out_shape=jax.ShapeDtypeStruct((init_fea2.shape[0], _D), jnp.float32),
    )(init_fea2, Wc.T, bc[None, :])
    x = jnp.concatenate([node_emb, cm], axis=0)

    nblk = _NT // _RB
    h_split = pl.pallas_call(
        _mlp_body,
        grid=(nblk,),
        in_specs=[
            pl.BlockSpec((_RB, _D), lambda i: (i, 0)),
            pl.BlockSpec((_D, W1.shape[0]), lambda i: (0, 0)),
            pl.BlockSpec((1, W1.shape[0]), lambda i: (0, 0)),
            pl.BlockSpec((W1.shape[0], _D), lambda i: (0, 0)),
            pl.BlockSpec((1, _D), lambda i: (0, 0)),
        ],
        out_specs=pl.BlockSpec((_NC, _RB, _DH), lambda i: (0, i, 0)),
        out_shape=jax.ShapeDtypeStruct((_NC, _NT, _DH), jnp.float32),
    )(x, W1.T, b1[None, :], W2.T, b2[None, :])
    h2 = h_split.reshape(_NC * _NT, _DH)

    e = adj_values.shape[0]
    epad = -(-e // (_NS * _G)) * (_NS * _G)
    rows = adj_indices[0].astype(jnp.int32)
    cols = adj_indices[1].astype(jnp.int32)
    vals = adj_values
    if epad != e:
        pad = epad - e
        rows = jnp.concatenate([rows, jnp.zeros((pad,), jnp.int32)])
        cols = jnp.concatenate([cols, jnp.zeros((pad,), jnp.int32)])
        vals = jnp.concatenate([vals, jnp.zeros((pad,), jnp.float32)])
    nb = epad // (_NS * _G)

    mesh = plsc.VectorSubcoreMesh(core_axis_name="c", subcore_axis_name="s",
                                  num_cores=_NC, num_subcores=_NS)
    sc = pl.kernel(
        functools.partial(_sc_body, nb),
        out_type=(
            jax.ShapeDtypeStruct((_NC * _NT, _DH), jnp.float32),
            jax.ShapeDtypeStruct((_NC * _NT, _DH), jnp.float32),
        ),
        mesh=mesh,
        scratch_types=[
            pltpu.VMEM((3, _G), jnp.int32),
            pltpu.VMEM((3, _G), jnp.int32),
            pltpu.VMEM((3, _G), jnp.float32),
            pltpu.VMEM((2, _G, _DH), jnp.float32),
            pltpu.VMEM_SHARED((_NT, _DH), jnp.float32),
            pltpu.SemaphoreType.DMA,
            pltpu.SemaphoreType.DMA,
            pltpu.SemaphoreType.DMA,
        ],
    )
    out2, _y = sc(h2, rows, cols, vals)
    o = out2.reshape(_NC, _NT, _DH)
    return jnp.concatenate([o[0], o[1]], axis=1)


# speedup vs baseline: 1.2740x; 1.2740x over previous
import functools

import jax
import jax.numpy as jnp
from jax import lax
from jax.experimental import pallas as pl
from jax.experimental.pallas import tpu as pltpu
from jax.experimental.pallas import tpu_sc as plsc

_NT = 12048
_D = 256
_DH = 128
_NC = 2
_NS = 16
_L = 16
_G = 128
_GPC = 2
_CH = _G * _GPC
_RGRP = 8
_RPT = 752
_RB = 2008


def _codemap_body(f2_ref, wct_ref, bc_ref, o_ref):
    o_ref[...] = (
        jnp.dot(f2_ref[...], wct_ref[...], preferred_element_type=jnp.float32)
        + bc_ref[...]
    )


def _mlp_body(x_ref, w1t_ref, b1_ref, w2t_ref, b2_ref, o_ref):
    h = jnp.dot(x_ref[...], w1t_ref[...], preferred_element_type=jnp.float32)
    h = h + b1_ref[...]
    h = jnp.where(h > 0, h, 0.1 * h)
    o = jnp.dot(h, w2t_ref[...], preferred_element_type=jnp.float32)
    o = o + b2_ref[...]
    o_ref[0] = o[:, :_DH]
    o_ref[1] = o[:, _DH:]


def _sc_body(nchunk, h_hbm, rowi_hbm, coli_hbm, vali_hbm, out_hbm, y_hbm,
             rowv, colv, valv, rows_v, acc, sem, ssem):
    c = lax.axis_index("c")
    s = lax.axis_index("s")
    cbias = c * _NT
    ebase = s * (nchunk * _CH)
    ngrp = nchunk * _GPC
    zerov = jnp.zeros((_L,), jnp.float32)

    def _zero_acc():
        def zbody(r, carry):
            for fb in range(_DH // _L):
                rows_v[r, pl.ds(fb * _L, _L)] = zerov
            return carry
        lax.fori_loop(0, _CH, zbody, 0)
        for off in range(0, _RPT, _CH):
            n = min(_CH, _RPT - off)
            pltpu.sync_copy(rows_v.at[pl.ds(0, n)],
                            acc.at[pl.ds(s * _RPT + off, n)])

        @pl.when(s == _NS - 1)
        def _():
            pltpu.sync_copy(rows_v.at[pl.ds(0, _NT - _NS * _RPT)],
                            acc.at[pl.ds(_NS * _RPT, _NT - _NS * _RPT)])

    def _pass(table_hbm, dst_hbm):
        def chunk(k, carry):
            e0 = ebase + k * _CH
            kper = _RGRP // _GPC

            @pl.when(lax.rem(k, kper) == 0)
            def _():
                g0 = s * ngrp + (k // kper) * _RGRP
                pltpu.sync_copy(rowi_hbm.at[pl.ds(g0, _RGRP)], rowv)

            pltpu.sync_copy(coli_hbm.at[pl.ds(e0, _CH)], colv)
            pltpu.sync_copy(vali_hbm.at[pl.ds(e0, _CH)], valv)

            def bias(t, cc):
                colv[pl.ds(t * _L, _L)] = colv[pl.ds(t * _L, _L)] + cbias
                return cc
            lax.fori_loop(0, _CH // _L, bias, 0)

            gat = [
                pltpu.async_copy(table_hbm.at[colv.at[pl.ds(j * _G, _G)]],
                                 rows_v.at[pl.ds(j * _G, _G)], sem)
                for j in range(_GPC)
            ]
            sca = []
            for j in range(_GPC):
                gat[j].wait()

                def scale(t, cc, j=j):
                    val16 = valv[pl.ds(j * _G + t * _L, _L)]
                    for u in range(_L):
                        r = j * _G + t * _L + u
                        v = val16[u]
                        for fb in range(_DH // _L):
                            rows_v[r, pl.ds(fb * _L, _L)] = (
                                rows_v[r, pl.ds(fb * _L, _L)] * v)
                    return cc
                lax.fori_loop(0, _G // _L, scale, 0)
                sca.append(pltpu.async_copy(
                    rows_v.at[pl.ds(j * _G, _G)],
                    acc.at[rowv.at[lax.rem(k, kper) * _GPC + j]],
                    ssem, add=True))
            for d in sca:
                d.wait()
            return carry
        lax.fori_loop(0, nchunk, chunk, 0)
        plsc.subcore_barrier()
        pltpu.sync_copy(acc.at[pl.ds(s * _RPT, _RPT)],
                        dst_hbm.at[pl.ds(cbias + s * _RPT, _RPT)])

        @pl.when(s == _NS - 1)
        def _():
            rem = _NT - _NS * _RPT
            pltpu.sync_copy(acc.at[pl.ds(_NS * _RPT, rem)],
                            dst_hbm.at[pl.ds(cbias + _NS * _RPT, rem)])

    _zero_acc()
    plsc.subcore_barrier()
    _pass(h_hbm, y_hbm)
    _zero_acc()
    plsc.subcore_barrier()
    _pass(y_hbm, out_hbm)


def kernel(seq_a, adj_indices, adj_values, node_emb, init_fea2, Wc, bc,
           W1, b1, W2, b2):
    del seq_a

    cm = pl.pallas_call(
        _codemap_body,
        out_shape=jax.ShapeDtypeStruct((init_fea2.shape[0], _D), jnp.float32),
    )(init_fea2, Wc.T, bc[None, :])
    x = jnp.concatenate([node_emb, cm], axis=0)

    nblk = _NT // _RB
    h_split = pl.pallas_call(
        _mlp_body,
        grid=(nblk,),
        in_specs=[
            pl.BlockSpec((_RB, _D), lambda i: (i, 0)),
            pl.BlockSpec((_D, W1.shape[0]), lambda i: (0, 0)),
            pl.BlockSpec((1, W1.shape[0]), lambda i: (0, 0)),
            pl.BlockSpec((W1.shape[0], _D), lambda i: (0, 0)),
            pl.BlockSpec((1, _D), lambda i: (0, 0)),
        ],
        out_specs=pl.BlockSpec((_NC, _RB, _DH), lambda i: (0, i, 0)),
        out_shape=jax.ShapeDtypeStruct((_NC, _NT, _DH), jnp.float32),
    )(x, W1.T, b1[None, :], W2.T, b2[None, :])
    h2 = h_split.reshape(_NC * _NT, _DH)

    e = adj_values.shape[0]
    epad = -(-e // (_NS * _CH)) * (_NS * _CH)
    rows = adj_indices[0].astype(jnp.int32)
    cols = adj_indices[1].astype(jnp.int32)
    vals = adj_values
    if epad != e:
        pad = epad - e
        rows = jnp.concatenate([rows, jnp.zeros((pad,), jnp.int32)])
        cols = jnp.concatenate([cols, jnp.zeros((pad,), jnp.int32)])
        vals = jnp.concatenate([vals, jnp.zeros((pad,), jnp.float32)])
    rows2 = rows.reshape(-1, _G)
    nchunk = epad // (_NS * _CH)

    mesh = plsc.VectorSubcoreMesh(core_axis_name="c", subcore_axis_name="s",
                                  num_cores=_NC, num_subcores=_NS)
    sc = pl.kernel(
        functools.partial(_sc_body, nchunk),
        out_type=(
            jax.ShapeDtypeStruct((_NC * _NT, _DH), jnp.float32),
            jax.ShapeDtypeStruct((_NC * _NT, _DH), jnp.float32),
        ),
        mesh=mesh,
        scratch_types=[
            pltpu.VMEM((_RGRP, _G), jnp.int32),
            pltpu.VMEM((_CH,), jnp.int32),
            pltpu.VMEM((_CH,), jnp.float32),
            pltpu.VMEM((_CH, _DH), jnp.float32),
            pltpu.VMEM_SHARED((_NT, _DH), jnp.float32),
            pltpu.SemaphoreType.DMA,
            pltpu.SemaphoreType.DMA,
        ],
    )
    out2, _y = sc(h2, rows2, cols, vals)
    o = out2.reshape(_NC, _NT, _DH)
    return jnp.concatenate([o[0], o[1]], axis=1)


# speedup vs baseline: 1.3994x; 1.0985x over previous
import functools

import jax
import jax.numpy as jnp
from jax import lax
from jax.experimental import pallas as pl
from jax.experimental.pallas import tpu as pltpu
from jax.experimental.pallas import tpu_sc as plsc

_NT = 12048
_D = 256
_DH = 128
_NC = 2
_NS = 16
_L = 16
_G = 128
_GPC = 2
_CH = _G * _GPC
_RGRP = 8
_RPT = 752
_RB = 2008


def _codemap_body(f2_ref, wct_ref, bc_ref, o_ref):
    o_ref[...] = (
        jnp.dot(f2_ref[...], wct_ref[...], preferred_element_type=jnp.float32)
        + bc_ref[...]
    )


def _mlp_body(x_ref, w1t_ref, b1_ref, w2t_ref, b2_ref, o_ref):
    h = jnp.dot(x_ref[...], w1t_ref[...], preferred_element_type=jnp.float32)
    h = h + b1_ref[...]
    h = jnp.where(h > 0, h, 0.1 * h)
    o = jnp.dot(h, w2t_ref[...], preferred_element_type=jnp.float32)
    o = o + b2_ref[...]
    o_ref[0] = o[:, :_DH]
    o_ref[1] = o[:, _DH:]


def _sc_body(nchunk, h_hbm, rowi_hbm, coli_hbm, vali_hbm, out_hbm, y_hbm,
             rowv, colv, valv, rows_v, acc, isem, gsem, ssem):
    c = lax.axis_index("c")
    s = lax.axis_index("s")
    cbias = c * _NT
    ebase = s * (nchunk * _CH)
    ngrp = nchunk * _GPC
    zerov = jnp.zeros((_L,), jnp.float32)

    def _zero_acc():
        def zbody(r, carry):
            for fb in range(_DH // _L):
                rows_v[r, pl.ds(fb * _L, _L)] = zerov
            return carry
        lax.fori_loop(0, _CH, zbody, 0)
        for off in range(0, _RPT, _CH):
            n = min(_CH, _RPT - off)
            pltpu.sync_copy(rows_v.at[pl.ds(0, n)],
                            acc.at[pl.ds(s * _RPT + off, n)])

        @pl.when(s == _NS - 1)
        def _():
            pltpu.sync_copy(rows_v.at[pl.ds(0, _NT - _NS * _RPT)],
                            acc.at[pl.ds(_NS * _RPT, _NT - _NS * _RPT)])

    def _pass(table_hbm, dst_hbm):
        kper = _RGRP // _GPC

        def sca_desc(k, j):
            return pltpu.make_async_copy(
                rows_v.at[pl.ds(j * _G, _G)],
                acc.at[rowv.at[lax.rem(k, kper) * _GPC + j]], ssem)

        pltpu.async_copy(coli_hbm.at[pl.ds(ebase, _CH)],
                         colv.at[pl.ds(0, _CH)], isem)

        def chunk(k, carry):
            e0 = ebase + k * _CH
            co = lax.rem(k, 2) * _CH

            pltpu.make_async_copy(coli_hbm.at[pl.ds(e0, _CH)],
                                  colv.at[pl.ds(co, _CH)], isem).wait()

            def bias(t, cc):
                colv[pl.ds(co + t * _L, _L)] = (
                    colv[pl.ds(co + t * _L, _L)] + cbias)
                return cc
            lax.fori_loop(0, _CH // _L, bias, 0)

            @pl.when(k >= 1)
            def _():
                sca_desc(k - 1, 0).wait()
                sca_desc(k - 1, 1).wait()

            @pl.when(lax.rem(k, kper) == 0)
            def _():
                g0 = s * ngrp + (k // kper) * _RGRP
                pltpu.sync_copy(rowi_hbm.at[pl.ds(g0, _RGRP)], rowv)

            gat = [
                pltpu.async_copy(
                    table_hbm.at[colv.at[pl.ds(co + j * _G, _G)]],
                    rows_v.at[pl.ds(j * _G, _G)], gsem)
                for j in range(_GPC)
            ]

            @pl.when(k < nchunk - 1)
            def _():
                pltpu.async_copy(
                    coli_hbm.at[pl.ds(e0 + _CH, _CH)],
                    colv.at[pl.ds(_CH - co, _CH)], isem)

            pltpu.sync_copy(vali_hbm.at[pl.ds(e0, _CH)], valv)

            for j in range(_GPC):
                gat[j].wait()

                def scale(t, cc, j=j):
                    val16 = valv[pl.ds(j * _G + t * _L, _L)]
                    for u in range(_L):
                        r = j * _G + t * _L + u
                        v = val16[u]
                        for fb in range(_DH // _L):
                            rows_v[r, pl.ds(fb * _L, _L)] = (
                                rows_v[r, pl.ds(fb * _L, _L)] * v)
                    return cc
                lax.fori_loop(0, _G // _L, scale, 0)
                pltpu.async_copy(
                    rows_v.at[pl.ds(j * _G, _G)],
                    acc.at[rowv.at[lax.rem(k, kper) * _GPC + j]],
                    ssem, add=True)
            return carry
        lax.fori_loop(0, nchunk, chunk, 0)
        sca_desc(nchunk - 1, 0).wait()
        sca_desc(nchunk - 1, 1).wait()
        plsc.subcore_barrier()
        pltpu.sync_copy(acc.at[pl.ds(s * _RPT, _RPT)],
                        dst_hbm.at[pl.ds(cbias + s * _RPT, _RPT)])

        @pl.when(s == _NS - 1)
        def _():
            rem = _NT - _NS * _RPT
            pltpu.sync_copy(acc.at[pl.ds(_NS * _RPT, rem)],
                            dst_hbm.at[pl.ds(cbias + _NS * _RPT, rem)])

    _zero_acc()
    plsc.subcore_barrier()
    _pass(h_hbm, y_hbm)
    _zero_acc()
    plsc.subcore_barrier()
    _pass(y_hbm, out_hbm)


def kernel(seq_a, adj_indices, adj_values, node_emb, init_fea2, Wc, bc,
           W1, b1, W2, b2):
    del seq_a

    cm = pl.pallas_call(
        _codemap_body,
        out_shape=jax.ShapeDtypeStruct((init_fea2.shape[0], _D), jnp.float32),
    )(init_fea2, Wc.T, bc[None, :])
    x = jnp.concatenate([node_emb, cm], axis=0)

    nblk = _NT // _RB
    h_split = pl.pallas_call(
        _mlp_body,
        grid=(nblk,),
        in_specs=[
            pl.BlockSpec((_RB, _D), lambda i: (i, 0)),
            pl.BlockSpec((_D, W1.shape[0]), lambda i: (0, 0)),
            pl.BlockSpec((1, W1.shape[0]), lambda i: (0, 0)),
            pl.BlockSpec((W1.shape[0], _D), lambda i: (0, 0)),
            pl.BlockSpec((1, _D), lambda i: (0, 0)),
        ],
        out_specs=pl.BlockSpec((_NC, _RB, _DH), lambda i: (0, i, 0)),
        out_shape=jax.ShapeDtypeStruct((_NC, _NT, _DH), jnp.float32),
    )(x, W1.T, b1[None, :], W2.T, b2[None, :])
    h2 = h_split.reshape(_NC * _NT, _DH)

    e = adj_values.shape[0]
    epad = -(-e // (_NS * _CH)) * (_NS * _CH)
    rows = adj_indices[0].astype(jnp.int32)
    cols = adj_indices[1].astype(jnp.int32)
    vals = adj_values
    if epad != e:
        pad = epad - e
        rows = jnp.concatenate([rows, jnp.zeros((pad,), jnp.int32)])
        cols = jnp.concatenate([cols, jnp.zeros((pad,), jnp.int32)])
        vals = jnp.concatenate([vals, jnp.zeros((pad,), jnp.float32)])
    rows2 = rows.reshape(-1, _G)
    nchunk = epad // (_NS * _CH)

    mesh = plsc.VectorSubcoreMesh(core_axis_name="c", subcore_axis_name="s",
                                  num_cores=_NC, num_subcores=_NS)
    sc = pl.kernel(
        functools.partial(_sc_body, nchunk),
        out_type=(
            jax.ShapeDtypeStruct((_NC * _NT, _DH), jnp.float32),
            jax.ShapeDtypeStruct((_NC * _NT, _DH), jnp.float32),
        ),
        mesh=mesh,
        scratch_types=[
            pltpu.VMEM((_RGRP, _G), jnp.int32),
            pltpu.VMEM((2 * _CH,), jnp.int32),
            pltpu.VMEM((_CH,), jnp.float32),
            pltpu.VMEM((_CH, _DH), jnp.float32),
            pltpu.VMEM_SHARED((_NT, _DH), jnp.float32),
            pltpu.SemaphoreType.DMA,
            pltpu.SemaphoreType.DMA,
            pltpu.SemaphoreType.DMA,
        ],
    )
    out2, _y = sc(h2, rows2, cols, vals)
    o = out2.reshape(_NC, _NT, _DH)
    return jnp.concatenate([o[0], o[1]], axis=1)
